# baseline (speedup 1.0000x reference)
import functools

import jax
import jax.numpy as jnp
import numpy as np
from jax import lax
from jax.experimental import pallas as pl
from jax.experimental.pallas import tpu as pltpu
from jax.experimental.pallas import tpu_sc as plsc

N_NODES = 10000
N_EDGES = 320000
D_FEAT = 128

NC = 2
NS = 16
NW = NC * NS
CHUNK = 128
N_CHUNKS = 80
E_PER_W = N_CHUNKS * CHUNK
E_PAD = NW * E_PER_W
N_PAD = 10112
ROWS_PER_TILE = N_PAD // NS

_PERM = np.zeros(D_FEAT, dtype=np.int32)
for _l in range(D_FEAT // 32):
    for _t in range(16):
        _PERM[32 * _l + 2 * _t] = 32 * _l + _t
        _PERM[32 * _l + 2 * _t + 1] = 32 * _l + 16 + _t


def _sc_partials(xu, row_idx, col_idx):
    mesh = plsc.VectorSubcoreMesh(core_axis_name="c", subcore_axis_name="s")

    @functools.partial(
        pl.kernel,
        mesh=mesh,
        compiler_params=pltpu.CompilerParams(use_tc_tiling_on_sc=False),
        out_type=jax.ShapeDtypeStruct((NC, N_PAD, D_FEAT), jnp.float32),
        scratch_types=[
            pltpu.VMEM((2, CHUNK), jnp.int32),
            pltpu.VMEM((4, CHUNK), jnp.int32),
            pltpu.VMEM((CHUNK, D_FEAT // 2), jnp.int32),
            pltpu.VMEM((CHUNK, D_FEAT // 2), jnp.int32),
            pltpu.VMEM((CHUNK, D_FEAT), jnp.float32),
            pltpu.VMEM((CHUNK, D_FEAT), jnp.float32),
            pltpu.VMEM_SHARED((N_PAD, D_FEAT), jnp.float32),
            pltpu.SemaphoreType.DMA,
            pltpu.SemaphoreType.DMA,
            pltpu.SemaphoreType.DMA,
            pltpu.SemaphoreType.DMA,
            pltpu.SemaphoreType.DMA,
            pltpu.SemaphoreType.DMA,
            pltpu.SemaphoreType.DMA,
        ],
    )
    def k(x_hbm, row_hbm, col_hbm, out_hbm,
          rowi, coli, m16a, m16b, msgfa, msgfb, acc,
          semg_a, semg_b, sems_a, sems_b, semr, semc_a, semc_b):
        c = lax.axis_index("c")
        s = lax.axis_index("s")
        wid = s * NC + c
        r0 = s * ROWS_PER_TILE
        m16 = (m16a, m16b)
        msgf = (msgfa, msgfb)
        semg = (semg_a, semg_b)
        sems = (sems_a, sems_b)
        semc = (semc_a, semc_b)

        zvec = jnp.zeros((16,), jnp.float32)

        def zbody(i, carry):
            for l in range(D_FEAT // 16):
                msgfa[i, pl.ds(l * 16, 16)] = zvec
            return carry

        lax.fori_loop(0, CHUNK, zbody, 0)
        for t in range(ROWS_PER_TILE // CHUNK):
            pltpu.sync_copy(msgfa, acc.at[pl.ds(r0 + t * CHUNK, CHUNK)])
        rem = ROWS_PER_TILE % CHUNK
        if rem:
            pltpu.sync_copy(msgfa.at[pl.ds(0, rem)],
                            acc.at[pl.ds(r0 + ROWS_PER_TILE - rem, rem)])
        plsc.subcore_barrier()

        pltpu.sync_copy(row_hbm.at[wid, 0], rowi.at[0])
        pltpu.sync_copy(row_hbm.at[wid, 1], rowi.at[1])
        pltpu.async_copy(x_hbm.at[rowi.at[0]], m16a, semg_a)
        pltpu.async_copy(x_hbm.at[rowi.at[1]], m16b, semg_b)
        pltpu.async_copy(col_hbm.at[wid, 0], coli.at[0], semc_b)
        pltpu.async_copy(col_hbm.at[wid, 1], coli.at[1], semc_b)

        hi16 = jnp.int32(-65536)

        def convert(src, dst):
            def cbody(r4, carry):
                for dr in range(4):
                    r = r4 * 4 + dr
                    for l in range(D_FEAT // 32):
                        m = src[r, pl.ds(l * 16, 16)]
                        even = lax.bitcast_convert_type(m << 16, jnp.float32)
                        odd = lax.bitcast_convert_type(m & hi16, jnp.float32)
                        dst[r, pl.ds(l * 32, 16)] = even
                        dst[r, pl.ds(l * 32 + 16, 16)] = odd
                return carry

            lax.fori_loop(0, CHUNK // 4, cbody, 0)

        def chunk_step(i2, off):
            j = 4 * i2 + off
            p = off % 2
            pltpu.make_async_copy(x_hbm.at[rowi.at[p]], m16[p], semg[p]).wait()

            @pl.when(j + 2 < N_CHUNKS)
            def _():
                pltpu.async_copy(row_hbm.at[wid, j + 2], rowi.at[p], semr)

            @pl.when(j >= 2)
            def _():
                pltpu.make_async_copy(msgf[p], acc.at[coli.at[off]],
                                      sems[p]).wait()

            @pl.when(j + 2 < N_CHUNKS)
            def _():
                pltpu.async_copy(col_hbm.at[wid, j + 2],
                                 coli.at[(off + 2) % 4],
                                 semc[0] if off < 2 else semc[1])

            convert(m16[p], msgf[p])
            pltpu.make_async_copy(col_hbm.at[wid, j], coli.at[off],
                                  semc[1] if off < 2 else semc[0]).wait()
            pltpu.async_copy(msgf[p], acc.at[coli.at[off]], sems[p], add=True)

            @pl.when(j + 2 < N_CHUNKS)
            def _():
                pltpu.make_async_copy(row_hbm.at[wid, j + 2], rowi.at[p],
                                      semr).wait()
                pltpu.async_copy(x_hbm.at[rowi.at[p]], m16[p], semg[p])

        def body(i2, carry):
            for off in range(4):
                chunk_step(i2, off)
            return carry

        lax.fori_loop(0, N_CHUNKS // 4, body, 0)
        pltpu.make_async_copy(msgfa, acc.at[coli.at[2]], sems_a).wait()
        pltpu.make_async_copy(msgfb, acc.at[coli.at[3]], sems_b).wait()

        plsc.subcore_barrier()
        pltpu.sync_copy(acc.at[pl.ds(r0, ROWS_PER_TILE)],
                        out_hbm.at[c, pl.ds(r0, ROWS_PER_TILE)])

    return k(xu, row_idx, col_idx)


def _tc_add(partials):
    blk = 2000

    def body(p_ref, o_ref):
        o_ref[...] = p_ref[0] + p_ref[1]

    return pl.pallas_call(
        body,
        grid=(N_NODES // blk,),
        in_specs=[pl.BlockSpec((NC, blk, D_FEAT), lambda i: (0, i, 0))],
        out_specs=pl.BlockSpec((blk, D_FEAT), lambda i: (i, 0)),
        out_shape=jax.ShapeDtypeStruct((N_NODES, D_FEAT), jnp.float32),
    )(partials)


def kernel(graph_or_x, edge_index):
    x = graph_or_x.astype(jnp.float32)
    xp = x[:, _PERM].astype(jnp.bfloat16)
    xu = jax.lax.bitcast_convert_type(
        xp.reshape(N_NODES, D_FEAT // 2, 2), jnp.int32)
    ei = edge_index.astype(jnp.int32)
    n_extra = E_PAD - N_EDGES
    pad = jnp.arange(n_extra, dtype=jnp.int32)
    row = jnp.concatenate([ei[0], pad % N_NODES])
    col = jnp.concatenate([ei[1], N_NODES + pad % (N_PAD - N_NODES)])
    row = row.reshape(NW, N_CHUNKS, CHUNK)
    col = col.reshape(NW, N_CHUNKS, CHUNK)
    partials = _sc_partials(xu, row, col)
    return _tc_add(partials)


# speedup vs baseline: 1.9415x; 1.9415x over previous
import functools

import jax
import jax.numpy as jnp
from jax import lax
from jax.experimental import pallas as pl
from jax.experimental.pallas import tpu as pltpu
from jax.experimental.pallas import tpu_sc as plsc

N_NODES = 10000
N_EDGES = 320000
D_FEAT = 128

NC = 2
NS = 16
NW = NC * NS
CHUNK = 128
N_CHUNKS = 80
HALF = N_CHUNKS // 2
E_PER_W = N_CHUNKS * CHUNK
E_PAD = NW * E_PER_W
N_PAD = 10240
ROWS_PER_TILE = N_PAD // NS


def _sc_partials(x, row_idx, col_idx):
    mesh = plsc.VectorSubcoreMesh(core_axis_name="c", subcore_axis_name="s")

    @functools.partial(
        pl.kernel,
        mesh=mesh,
        out_type=jax.ShapeDtypeStruct((NC, N_PAD, D_FEAT), jnp.float32),
        scratch_types=[
            pltpu.VMEM((HALF, CHUNK), jnp.int32),
            pltpu.VMEM((HALF, CHUNK), jnp.int32),
            pltpu.VMEM((CHUNK, D_FEAT), jnp.float32),
            pltpu.VMEM((CHUNK, D_FEAT), jnp.float32),
            pltpu.VMEM_SHARED((N_PAD, D_FEAT), jnp.float32),
            pltpu.SemaphoreType.DMA,
            pltpu.SemaphoreType.DMA,
            pltpu.SemaphoreType.DMA,
            pltpu.SemaphoreType.DMA,
        ],
    )
    def k(x_hbm, row_hbm, col_hbm, out_hbm,
          row_v, col_v, msg_a, msg_b, acc, sem_a, sem_b, sem_sa, sem_sb):
        c = lax.axis_index("c")
        s = lax.axis_index("s")
        wid = s * NC + c
        r0 = s * ROWS_PER_TILE
        zvec = jnp.zeros((16,), jnp.float32)

        def zbody(i, carry):
            for l in range(D_FEAT // 16):
                msg_a[i, pl.ds(l * 16, 16)] = zvec
            return carry

        lax.fori_loop(0, CHUNK, zbody, 0)
        for t in range(ROWS_PER_TILE // CHUNK):
            pltpu.sync_copy(msg_a, acc.at[pl.ds(r0 + t * CHUNK, CHUNK)])
        plsc.subcore_barrier()

        for h in range(2):
            pltpu.sync_copy(row_hbm.at[wid, h], row_v)
            pltpu.sync_copy(col_hbm.at[wid, h], col_v)
            pltpu.async_copy(x_hbm.at[row_v.at[0]], msg_a, sem_a)

            def body(i, carry):
                ja = 2 * i
                jb = 2 * i + 1
                pltpu.async_copy(x_hbm.at[row_v.at[jb]], msg_b, sem_b)
                pltpu.make_async_copy(x_hbm.at[row_v.at[ja]], msg_a, sem_a).wait()
                pltpu.sync_copy(msg_a, acc.at[col_v.at[ja]], add=True)

                @pl.when(jb + 1 < HALF)
                def _():
                    pltpu.async_copy(x_hbm.at[row_v.at[jb + 1]], msg_a, sem_a)

                pltpu.make_async_copy(x_hbm.at[row_v.at[jb]], msg_b, sem_b).wait()
                pltpu.sync_copy(msg_b, acc.at[col_v.at[jb]], add=True)
                return carry

            lax.fori_loop(0, HALF // 2, body, 0)

        plsc.subcore_barrier()
        pltpu.sync_copy(acc.at[pl.ds(r0, ROWS_PER_TILE)],
                        out_hbm.at[c, pl.ds(r0, ROWS_PER_TILE)])

    return k(x, row_idx, col_idx)


def _tc_add(partials):
    blk = 2000

    def body(p_ref, o_ref):
        o_ref[...] = p_ref[0] + p_ref[1]

    return pl.pallas_call(
        body,
        grid=(N_NODES // blk,),
        in_specs=[pl.BlockSpec((NC, blk, D_FEAT), lambda i: (0, i, 0))],
        out_specs=pl.BlockSpec((blk, D_FEAT), lambda i: (i, 0)),
        out_shape=jax.ShapeDtypeStruct((N_NODES, D_FEAT), jnp.float32),
    )(partials)


def kernel(graph_or_x, edge_index):
    x = graph_or_x.astype(jnp.float32)
    ei = edge_index.astype(jnp.int32)
    n_extra = E_PAD - N_EDGES
    pad = jnp.arange(n_extra, dtype=jnp.int32)
    row = jnp.concatenate([ei[0], pad % N_NODES])
    col = jnp.concatenate([ei[1], N_NODES + pad % (N_PAD - N_NODES)])
    row = row.reshape(NW, 2, HALF, CHUNK)
    col = col.reshape(NW, 2, HALF, CHUNK)
    partials = _sc_partials(x, row, col)
    return _tc_add(partials)
